# baseline (speedup 1.0000x reference)
import dataclasses
import functools

import jax
import jax.numpy as jnp
from jax import lax
from jax.experimental import pallas as pl
from jax.experimental.pallas import tpu as pltpu
from jax.experimental.pallas import tpu_sc as plsc

_L = 16
_B = 64
_BT = 512
_SPLIT = 2.0

_IBIG = 2**31 - 1


def _row_max_argmax(load_chunk, nchunks, iota):
    def scan_range(j0, j1):
        m = load_chunk(j0)
        idx = iota + j0 * _L
        c = iota + (j0 + 1) * _L
        for j in range(j0 + 1, j1):
            v = load_chunk(j)
            gt = v > m
            m = jnp.where(gt, v, m)
            idx = jnp.where(gt, c, idx)
            c = c + _L
        return m, idx

    h = nchunks // 2
    m1, i1 = scan_range(0, h)
    m2, i2 = scan_range(h, nchunks)
    gt = m2 > m1
    m = jnp.where(gt, m2, m1)
    idx = jnp.where(gt, i2, i1)
    mx = jnp.max(m)
    amax = jnp.min(jnp.where(m == mx, idx, _IBIG))
    return mx, amax


def _sc_block_body(x_vmem, mx_vmem, ax_vmem):
    wx = x_vmem.shape[2]
    iota = lax.iota(jnp.int32, _L)
    zf = jnp.zeros((_L,), jnp.float32)

    for g in range(_B // _L):
        def row_body(i2, accs, g=g):
            mxa, axa = accs
            for u in range(2):
                i = 2 * i2 + u
                r = g * _L + i
                mx, ax = _row_max_argmax(
                    lambda j: x_vmem[0, r, pl.ds(j * _L, _L)],
                    wx // _L, iota)
                lane = iota == i
                mxa = jnp.where(lane, mx, mxa)
                axa = jnp.where(lane, ax.astype(jnp.float32), axa)
            return (mxa, axa)

        mxa, axa = lax.fori_loop(0, _L // 2, row_body, (zf, zf))
        mx_vmem[g, :] = mxa
        ax_vmem[g, :] = axa


@jax.jit
def _sc_decode_x(xt):
    k, n, wx = xt.shape
    mesh = plsc.VectorSubcoreMesh(core_axis_name="core",
                                  subcore_axis_name="subcore")
    out_t = [jax.ShapeDtypeStruct((k * n // _L, _L), jnp.float32),
             jax.ShapeDtypeStruct((k * n // _L, _L), jnp.float32)]
    cp = pltpu.CompilerParams(use_tc_tiling_on_sc=True)
    if "needs_layout_passes" in pltpu.CompilerParams.__dataclass_fields__:
        cp = dataclasses.replace(cp, needs_layout_passes=False)

    nblk = n // _B

    @functools.partial(pl.kernel, out_type=out_t, mesh=mesh,
                       compiler_params=cp)
    def kern(x_hbm, mx_hbm, ax_hbm):
        pltpu.emit_pipeline(
            _sc_block_body,
            grid=(k, nblk),
            in_specs=[
                pl.BlockSpec((1, _B, wx), lambda kk, i: (kk, i, 0)),
            ],
            out_specs=[
                pl.BlockSpec((_B // _L, _L),
                             lambda kk, i: (kk * nblk + i, 0)),
                pl.BlockSpec((_B // _L, _L),
                             lambda kk, i: (kk * nblk + i, 0)),
            ],
            core_axis_name=("core", "subcore"),
            dimension_semantics=(pltpu.PARALLEL, pltpu.PARALLEL),
        )(x_hbm, mx_hbm, ax_hbm)

    return kern(xt)


def _tc_block_body(y_ref, my_ref, ay_ref):
    yb = y_ref[...]
    mx = jnp.max(yb, axis=-1)
    eq = yb == mx[:, :, None]
    ii = lax.broadcasted_iota(jnp.int32, yb.shape, 2)
    ay = jnp.min(jnp.where(eq, ii, _IBIG), axis=-1)
    my_ref[...] = mx
    ay_ref[...] = ay.astype(jnp.float32)


@jax.jit
def _tc_decode_y(yt):
    k, n, wy = yt.shape
    out_t = [jax.ShapeDtypeStruct((k, n), jnp.float32),
             jax.ShapeDtypeStruct((k, n), jnp.float32)]
    grid = (n // _BT,)
    return pl.pallas_call(
        _tc_block_body,
        grid=grid,
        in_specs=[pl.BlockSpec((k, _BT, wy), lambda i: (0, i, 0))],
        out_specs=[pl.BlockSpec((k, _BT), lambda i: (0, i)),
                   pl.BlockSpec((k, _BT), lambda i: (0, i))],
        out_shape=out_t,
        compiler_params=pltpu.CompilerParams(
            dimension_semantics=("arbitrary",)),
    )(yt)


def kernel(simcc_x, simcc_y):
    n, k, _ = simcc_x.shape
    xt = jnp.transpose(simcc_x, (1, 0, 2))
    yt = jnp.transpose(simcc_y, (1, 0, 2))
    mxo, axo = _sc_decode_x(xt)
    my, ay = _tc_decode_y(yt)
    mx = mxo.reshape(k, n)
    ax = axo.reshape(k, n)
    val = jnp.minimum(mx, my)
    neg = val <= 0.0
    half = jnp.float32(1.0 / _SPLIT)
    kx = jnp.where(neg, jnp.float32(-1.0), ax) * half
    ky = jnp.where(neg, jnp.float32(-1.0), ay) * half
    keypoints = jnp.stack([kx, ky], axis=-1).transpose(1, 0, 2)
    scores = val.transpose(1, 0)
    return (keypoints, scores)


# speedup vs baseline: 1.0426x; 1.0426x over previous
import dataclasses
import functools

import jax
import jax.numpy as jnp
from jax import lax
from jax.experimental import pallas as pl
from jax.experimental.pallas import tpu as pltpu
from jax.experimental.pallas import tpu_sc as plsc

_L = 16
_B = 64
_BT = 512
_SPLIT = 2.0

_IBIG = 2**31 - 1


def _row_max_argmax(load_chunk, nchunks, iota):
    def scan_range(j0, j1):
        m = load_chunk(j0)
        idx = iota + j0 * _L
        c = iota + (j0 + 1) * _L
        for j in range(j0 + 1, j1):
            v = load_chunk(j)
            gt = v > m
            m = jnp.where(gt, v, m)
            idx = jnp.where(gt, c, idx)
            c = c + _L
        return m, idx

    h = nchunks // 2
    m1, i1 = scan_range(0, h)
    m2, i2 = scan_range(h, nchunks)
    gt = m2 > m1
    m = jnp.where(gt, m2, m1)
    idx = jnp.where(gt, i2, i1)
    mx = jnp.max(m)
    amax = jnp.min(jnp.where(m == mx, idx, _IBIG))
    return mx, amax


def _sc_block_body(x_vmem, mx_vmem, ax_vmem):
    wx = x_vmem.shape[2]
    iota = lax.iota(jnp.int32, _L)
    zf = jnp.zeros((_L,), jnp.float32)
    zi = jnp.zeros((_L,), jnp.int32)

    for g in range(_B // _L):
        def row_body(i2, accs, g=g):
            mxa, axa = accs
            for u in range(2):
                i = 2 * i2 + u
                r = g * _L + i
                mx, ax = _row_max_argmax(
                    lambda j: x_vmem[0, r, pl.ds(j * _L, _L)],
                    wx // _L, iota)
                lane = iota == i
                mxa = jnp.where(lane, mx, mxa)
                axa = jnp.where(lane, ax, axa)
            return (mxa, axa)

        mxa, axa = lax.fori_loop(0, _L // 2, row_body, (zf, zi))
        mx_vmem[g, :] = mxa
        ax_vmem[g, :] = axa


@jax.jit
def _sc_decode_x(xt):
    k, n, wx = xt.shape
    mesh = plsc.VectorSubcoreMesh(core_axis_name="core",
                                  subcore_axis_name="subcore")
    out_t = [jax.ShapeDtypeStruct((k * n // _L, _L), jnp.float32),
             jax.ShapeDtypeStruct((k * n // _L, _L), jnp.int32)]
    cp = pltpu.CompilerParams(use_tc_tiling_on_sc=True)
    if "needs_layout_passes" in pltpu.CompilerParams.__dataclass_fields__:
        cp = dataclasses.replace(cp, needs_layout_passes=False)

    nblk = n // _B

    @functools.partial(pl.kernel, out_type=out_t, mesh=mesh,
                       compiler_params=cp)
    def kern(x_hbm, mx_hbm, ax_hbm):
        pltpu.emit_pipeline(
            _sc_block_body,
            grid=(k, nblk),
            in_specs=[
                pl.BlockSpec((1, _B, wx), lambda kk, i: (kk, i, 0)),
            ],
            out_specs=[
                pl.BlockSpec((_B // _L, _L),
                             lambda kk, i: (kk * nblk + i, 0)),
                pl.BlockSpec((_B // _L, _L),
                             lambda kk, i: (kk * nblk + i, 0)),
            ],
            core_axis_name=("core", "subcore"),
            dimension_semantics=(pltpu.PARALLEL, pltpu.PARALLEL),
        )(x_hbm, mx_hbm, ax_hbm)

    return kern(xt)


def _tc_block_body(y_ref, my_ref, ay_ref):
    yb = y_ref[...]
    mx = jnp.max(yb, axis=-1)
    eq = yb == mx[:, :, None]
    ii = lax.broadcasted_iota(jnp.int32, yb.shape, 2)
    ay = jnp.min(jnp.where(eq, ii, _IBIG), axis=-1)
    my_ref[...] = mx
    ay_ref[...] = ay


@jax.jit
def _tc_decode_y(yt):
    k, n, wy = yt.shape
    out_t = [jax.ShapeDtypeStruct((k, n), jnp.float32),
             jax.ShapeDtypeStruct((k, n), jnp.int32)]
    grid = (n // _BT,)
    return pl.pallas_call(
        _tc_block_body,
        grid=grid,
        in_specs=[pl.BlockSpec((k, _BT, wy), lambda i: (0, i, 0))],
        out_specs=[pl.BlockSpec((k, _BT), lambda i: (0, i)),
                   pl.BlockSpec((k, _BT), lambda i: (0, i))],
        out_shape=out_t,
        compiler_params=pltpu.CompilerParams(
            dimension_semantics=("arbitrary",)),
    )(yt)


def kernel(simcc_x, simcc_y):
    n, k, _ = simcc_x.shape
    xt = jnp.transpose(simcc_x, (1, 0, 2))
    yt = jnp.transpose(simcc_y, (1, 0, 2))
    mxo, axo = _sc_decode_x(xt)
    my, ay = _tc_decode_y(yt)
    mx = mxo.reshape(k, n)
    ax = axo.reshape(k, n)
    val = jnp.minimum(mx, my)
    neg = val <= 0.0
    half = jnp.float32(1.0 / _SPLIT)
    kx = jnp.where(neg, jnp.float32(-1.0), ax.astype(jnp.float32)) * half
    ky = jnp.where(neg, jnp.float32(-1.0), ay.astype(jnp.float32)) * half
    keypoints = jnp.stack([kx, ky], axis=-1).transpose(1, 0, 2)
    scores = val.transpose(1, 0)
    return (keypoints, scores)
